# baseline (speedup 1.0000x reference)
import functools

import jax
import jax.numpy as jnp
from jax.experimental import pallas as pl
from jax.experimental.pallas import tpu as pltpu
from jax.experimental.pallas import tpu_sc as plsc

_NUM_ENTITIES = 1000000
_EMBED = 64
_BATCH = 16384
_H1, _H2 = 512, 256
_FEAT = 7 * _EMBED

_NUM_IDX = 3 * _BATCH
_GATHER_WINDOW = 128
_BLOCK_B = 2048


_SC_CORES = 2
_SC_SUBCORES = 16
_NW = _SC_CORES * _SC_SUBCORES
_B_PER_W = _NUM_IDX // _NW


def _sc_gather(table, idx):
    mesh = plsc.VectorSubcoreMesh(core_axis_name="c", subcore_axis_name="s")

    @functools.partial(
        pl.kernel,
        mesh=mesh,
        out_type=jax.ShapeDtypeStruct((_NUM_IDX, _EMBED), jnp.float32),
        scratch_types=[
            pltpu.VMEM((_B_PER_W,), jnp.int32),
            pltpu.SemaphoreType.DMA,
        ],
    )
    def gather_kernel(table_hbm, idx_hbm, out_hbm, idx_v, sem):
        wid = jax.lax.axis_index("s") * _SC_CORES + jax.lax.axis_index("c")
        base = wid * _B_PER_W
        pltpu.sync_copy(idx_hbm.at[pl.ds(base, _B_PER_W)], idx_v)

        @pl.loop(0, _B_PER_W, step=16)
        def _fire(i):
            vec = idx_v[pl.ds(i, 16)]
            for j in range(16):
                pltpu.async_copy(
                    table_hbm.at[vec[j]], out_hbm.at[base + i + j], sem
                )

        @pl.loop(0, _B_PER_W)
        def _drain(i):
            pltpu.make_async_copy(table_hbm.at[0], out_hbm.at[base], sem).wait()

    return gather_kernel(table, idx)


def _mlp_kernel(e_ref, w1_ref, b1_ref, w2_ref, b2_ref, wout_ref, bout_ref, o_ref):
    e1 = e_ref[0]
    e2 = e_ref[1]
    et = e_ref[2]
    feats = jnp.concatenate(
        [e1, e2, et, e1 * e2, e1 * et, e2 * et, e1 - e2], axis=-1
    )
    h = jnp.dot(feats, w1_ref[...], preferred_element_type=jnp.float32)
    h = jnp.maximum(h + b1_ref[...], 0.0)
    h = jnp.dot(h, w2_ref[...], preferred_element_type=jnp.float32)
    h = jnp.maximum(h + b2_ref[...], 0.0)
    out = jnp.dot(h, wout_ref[...], preferred_element_type=jnp.float32)
    o_ref[...] = out + bout_ref[0, 0]


def kernel(c1, c2, target, table, W1, b1, W2, b2, Wout, bout):
    idx = jnp.concatenate([c1, c2, target], axis=0).astype(jnp.int32)
    e = _sc_gather(table, idx).reshape(3, _BATCH, _EMBED)

    out = pl.pallas_call(
        _mlp_kernel,
        grid=(_BATCH // _BLOCK_B,),
        in_specs=[
            pl.BlockSpec((3, _BLOCK_B, _EMBED), lambda i: (0, i, 0)),
            pl.BlockSpec((_FEAT, _H1), lambda i: (0, 0)),
            pl.BlockSpec((1, _H1), lambda i: (0, 0)),
            pl.BlockSpec((_H1, _H2), lambda i: (0, 0)),
            pl.BlockSpec((1, _H2), lambda i: (0, 0)),
            pl.BlockSpec((_H2, 1), lambda i: (0, 0)),
            pl.BlockSpec((1, 1), lambda i: (0, 0)),
        ],
        out_specs=pl.BlockSpec((_BLOCK_B, 1), lambda i: (i, 0)),
        out_shape=jax.ShapeDtypeStruct((_BATCH, 1), jnp.float32),
    )(
        e,
        W1,
        b1.reshape(1, _H1),
        W2,
        b2.reshape(1, _H2),
        Wout,
        bout.reshape(1, 1),
    )
    return out[:, 0]


# speedup vs baseline: 1.6348x; 1.6348x over previous
import functools

import jax
import jax.numpy as jnp
from jax.experimental import pallas as pl
from jax.experimental.pallas import tpu as pltpu
from jax.experimental.pallas import tpu_sc as plsc

_NUM_ENTITIES = 1000000
_EMBED = 64
_BATCH = 16384
_H1, _H2 = 512, 256
_FEAT = 7 * _EMBED

_NUM_IDX = 3 * _BATCH
_GATHER_WINDOW = 128
_BLOCK_B = 2048


_SC_CORES = 2
_SC_SUBCORES = 16
_NW = _SC_CORES * _SC_SUBCORES
_B_PER_W = _NUM_IDX // _NW


_WIN = 128
_WIDE = 2 * _EMBED


def _sc_gather_wide(tablew, idxw):
    mesh = plsc.VectorSubcoreMesh(core_axis_name="c", subcore_axis_name="s")

    @functools.partial(
        pl.kernel,
        mesh=mesh,
        out_type=jax.ShapeDtypeStruct((_NUM_IDX, _WIDE), jnp.float32),
    )
    def gather_kernel(tbl_hbm, idx_hbm, out_hbm):
        def body(i_vmem, o_vmem):
            pltpu.sync_copy(tbl_hbm.at[i_vmem.at[0]], o_vmem)

        pltpu.emit_pipeline(
            body,
            grid=(_NUM_IDX // _WIN,),
            in_specs=[pl.BlockSpec((1, _WIN), lambda i: (0, i))],
            out_specs=[pl.BlockSpec((_WIN, _WIDE), lambda i: (i, 0))],
            core_axis_name=("c", "s"),
            dimension_semantics=(pltpu.PARALLEL,),
        )(idx_hbm, out_hbm)

    return gather_kernel(tablew, idxw.reshape(1, _NUM_IDX))


def _mlp_kernel(
    g_ref, p_ref, w1_ref, b1_ref, w2_ref, b2_ref, wout_ref, bout_ref, o_ref
):
    def pick(k):
        g = g_ref[k]
        p = p_ref[k]
        return jnp.where(p == 0, g[:, :_EMBED], g[:, _EMBED:])

    e1 = pick(0)
    e2 = pick(1)
    et = pick(2)
    feats = jnp.concatenate(
        [e1, e2, et, e1 * e2, e1 * et, e2 * et, e1 - e2], axis=-1
    )
    h = jnp.dot(feats, w1_ref[...], preferred_element_type=jnp.float32)
    h = jnp.maximum(h + b1_ref[...], 0.0)
    h = jnp.dot(h, w2_ref[...], preferred_element_type=jnp.float32)
    h = jnp.maximum(h + b2_ref[...], 0.0)
    out = jnp.dot(h, wout_ref[...], preferred_element_type=jnp.float32)
    o_ref[...] = out + bout_ref[0, 0]


def kernel(c1, c2, target, table, W1, b1, W2, b2, Wout, bout):
    idx = jnp.concatenate([c1, c2, target], axis=0).astype(jnp.int32)
    idxw = idx >> 1
    par = (idx & 1).reshape(3, _BATCH, 1)
    tablew = table.reshape(_NUM_ENTITIES // 2, _WIDE)
    g = _sc_gather_wide(tablew, idxw).reshape(3, _BATCH, _WIDE)

    out = pl.pallas_call(
        _mlp_kernel,
        grid=(_BATCH // _BLOCK_B,),
        in_specs=[
            pl.BlockSpec((3, _BLOCK_B, _WIDE), lambda i: (0, i, 0)),
            pl.BlockSpec((3, _BLOCK_B, 1), lambda i: (0, i, 0)),
            pl.BlockSpec((_FEAT, _H1), lambda i: (0, 0)),
            pl.BlockSpec((1, _H1), lambda i: (0, 0)),
            pl.BlockSpec((_H1, _H2), lambda i: (0, 0)),
            pl.BlockSpec((1, _H2), lambda i: (0, 0)),
            pl.BlockSpec((_H2, 1), lambda i: (0, 0)),
            pl.BlockSpec((1, 1), lambda i: (0, 0)),
        ],
        out_specs=pl.BlockSpec((_BLOCK_B, 1), lambda i: (i, 0)),
        out_shape=jax.ShapeDtypeStruct((_BATCH, 1), jnp.float32),
    )(
        g,
        par,
        W1,
        b1.reshape(1, _H1),
        W2,
        b2.reshape(1, _H2),
        Wout,
        bout.reshape(1, 1),
    )
    return out[:, 0]


# speedup vs baseline: 1.6984x; 1.0389x over previous
import functools

import jax
import jax.numpy as jnp
from jax.experimental import pallas as pl
from jax.experimental.pallas import tpu as pltpu
from jax.experimental.pallas import tpu_sc as plsc

_NUM_ENTITIES = 1000000
_EMBED = 64
_BATCH = 16384
_H1, _H2 = 512, 256
_FEAT = 7 * _EMBED

_NUM_IDX = 3 * _BATCH
_GATHER_WINDOW = 128
_BLOCK_B = 2048


_SC_CORES = 2
_SC_SUBCORES = 16
_NW = _SC_CORES * _SC_SUBCORES
_B_PER_W = _NUM_IDX // _NW


_WIN = 128
_WIDE = 2 * _EMBED
_HALF = _NUM_ENTITIES // 2
_PACK_R = 4000


def _pack_kernel(lo_ref, hi_ref, o_ref):
    o_ref[:, :_EMBED] = lo_ref[...]
    o_ref[:, _EMBED:] = hi_ref[...]


def _tc_pack(table):
    return pl.pallas_call(
        _pack_kernel,
        grid=(_HALF // _PACK_R,),
        in_specs=[
            pl.BlockSpec((_PACK_R, _EMBED), lambda i: (i, 0)),
            pl.BlockSpec((_PACK_R, _EMBED), lambda i: (_HALF // _PACK_R + i, 0)),
        ],
        out_specs=pl.BlockSpec((_PACK_R, _WIDE), lambda i: (i, 0)),
        out_shape=jax.ShapeDtypeStruct((_HALF, _WIDE), jnp.float32),
        compiler_params=pltpu.CompilerParams(
            dimension_semantics=("parallel",)
        ),
    )(table, table)


def _sc_gather_wide(tablew, idxw):
    mesh = plsc.VectorSubcoreMesh(core_axis_name="c", subcore_axis_name="s")

    @functools.partial(
        pl.kernel,
        mesh=mesh,
        out_type=jax.ShapeDtypeStruct((_NUM_IDX, _WIDE), jnp.float32),
    )
    def gather_kernel(tbl_hbm, idx_hbm, out_hbm):
        def body(i_vmem, o_vmem):
            pltpu.sync_copy(tbl_hbm.at[i_vmem.at[0]], o_vmem)

        pltpu.emit_pipeline(
            body,
            grid=(_NUM_IDX // _WIN,),
            in_specs=[pl.BlockSpec((1, _WIN), lambda i: (0, i))],
            out_specs=[pl.BlockSpec((_WIN, _WIDE), lambda i: (i, 0))],
            core_axis_name=("c", "s"),
            dimension_semantics=(pltpu.PARALLEL,),
        )(idx_hbm, out_hbm)

    return gather_kernel(tablew, idxw.reshape(1, _NUM_IDX))


def _mlp_kernel(
    g_ref, p_ref, w1_ref, b1_ref, w2_ref, b2_ref, wout_ref, bout_ref, o_ref
):
    def pick(k):
        g = g_ref[k]
        p = p_ref[k]
        return jnp.where(p == 0, g[:, :_EMBED], g[:, _EMBED:])

    e1 = pick(0)
    e2 = pick(1)
    et = pick(2)
    feats = jnp.concatenate(
        [e1, e2, et, e1 * e2, e1 * et, e2 * et, e1 - e2], axis=-1
    )
    h = jnp.dot(feats, w1_ref[...], preferred_element_type=jnp.float32)
    h = jnp.maximum(h + b1_ref[...], 0.0)
    h = jnp.dot(h, w2_ref[...], preferred_element_type=jnp.float32)
    h = jnp.maximum(h + b2_ref[...], 0.0)
    out = jnp.dot(h, wout_ref[...], preferred_element_type=jnp.float32)
    o_ref[...] = out + bout_ref[0, 0]


def kernel(c1, c2, target, table, W1, b1, W2, b2, Wout, bout):
    idx = jnp.concatenate([c1, c2, target], axis=0).astype(jnp.int32)
    idxw = jnp.where(idx < _HALF, idx, idx - _HALF)
    par = (idx >= _HALF).astype(jnp.int32).reshape(3, _BATCH, 1)
    tablew = _tc_pack(table)
    g = _sc_gather_wide(tablew, idxw).reshape(3, _BATCH, _WIDE)

    out = pl.pallas_call(
        _mlp_kernel,
        grid=(_BATCH // _BLOCK_B,),
        in_specs=[
            pl.BlockSpec((3, _BLOCK_B, _WIDE), lambda i: (0, i, 0)),
            pl.BlockSpec((3, _BLOCK_B, 1), lambda i: (0, i, 0)),
            pl.BlockSpec((_FEAT, _H1), lambda i: (0, 0)),
            pl.BlockSpec((1, _H1), lambda i: (0, 0)),
            pl.BlockSpec((_H1, _H2), lambda i: (0, 0)),
            pl.BlockSpec((1, _H2), lambda i: (0, 0)),
            pl.BlockSpec((_H2, 1), lambda i: (0, 0)),
            pl.BlockSpec((1, 1), lambda i: (0, 0)),
        ],
        out_specs=pl.BlockSpec((_BLOCK_B, 1), lambda i: (i, 0)),
        out_shape=jax.ShapeDtypeStruct((_BATCH, 1), jnp.float32),
    )(
        g,
        par,
        W1,
        b1.reshape(1, _H1),
        W2,
        b2.reshape(1, _H2),
        Wout,
        bout.reshape(1, 1),
    )
    return out[:, 0]
